# Initial kernel scaffold; baseline (speedup 1.0000x reference)
import jax
import jax.numpy as jnp
from jax.experimental import pallas as pl


def kernel(index, bra_weight, ket_weight):
    raise NotImplementedError("write your pallas kernel here")


# speedup vs baseline: 1.1311x; 1.1311x over previous
import functools

import jax
import jax.numpy as jnp
from jax import lax
from jax.experimental import pallas as pl
from jax.experimental.pallas import tpu as pltpu
from jax.experimental.pallas import tpu_sc as plsc

NUM_ENTITIES = 1000000
D = 64
B = 4096
F = 100
TOT = B * F

NC, NS = 2, 16
NW = NC * NS
PER_W = TOT // NW
C = 128
NCHUNK = PER_W // C
NBUF = 4
NGROUP = NCHUNK // NBUF


@functools.partial(
    pl.kernel,
    out_type=(
        jax.ShapeDtypeStruct((TOT, D), jnp.float32),
        jax.ShapeDtypeStruct((TOT, D), jnp.float32),
    ),
    mesh=plsc.VectorSubcoreMesh(core_axis_name="c", subcore_axis_name="s"),
    compiler_params=pltpu.CompilerParams(use_tc_tiling_on_sc=False),
    scratch_types=[
        pltpu.VMEM((NCHUNK, C), jnp.int32),
        pltpu.VMEM((NBUF, C, D), jnp.float32),
        pltpu.VMEM((NBUF, C, D), jnp.float32),
        pltpu.SemaphoreType.DMA,
        pltpu.SemaphoreType.DMA,
    ],
)
def _bracket_gather(idx_hbm, bra_hbm, ket_hbm, bra_out, ket_out,
                    idx_v, bra_v, ket_v, gsem, ssem):
    wid = lax.axis_index("s") * NC + lax.axis_index("c")
    base = wid * PER_W
    pltpu.sync_copy(idx_hbm.at[wid], idx_v)

    def group(g, carry):
        start = g * NBUF
        gathers = []
        for b in range(NBUF):
            j = start + b
            gathers.append(
                pltpu.async_copy(bra_hbm.at[idx_v.at[j]], bra_v.at[b], gsem))
            gathers.append(
                pltpu.async_copy(ket_hbm.at[idx_v.at[j]], ket_v.at[b], gsem))
        for h in gathers:
            h.wait()
        stores = []
        for b in range(NBUF):
            j = start + b
            off = base + j * C
            stores.append(
                pltpu.async_copy(bra_v.at[b], bra_out.at[pl.ds(off, C)], ssem))
            stores.append(
                pltpu.async_copy(ket_v.at[b], ket_out.at[pl.ds(off, C)], ssem))
        for h in stores:
            h.wait()
        return carry

    lax.fori_loop(0, NGROUP, group, 0)


def kernel(index, bra_weight, ket_weight):
    idx = index.reshape(NW, NCHUNK, C).astype(jnp.int32)
    bra_flat, ket_flat = _bracket_gather(idx, bra_weight, ket_weight)
    return (bra_flat.reshape(B, F, D), ket_flat.reshape(B, F, D))


# speedup vs baseline: 1.1379x; 1.0060x over previous
import functools

import jax
import jax.numpy as jnp
from jax import lax
from jax.experimental import pallas as pl
from jax.experimental.pallas import tpu as pltpu
from jax.experimental.pallas import tpu_sc as plsc

NUM_ENTITIES = 1000000
D = 64
B = 4096
F = 100
TOT = B * F

NC, NS = 2, 16
NW = NC * NS
PER_W = TOT // NW
C = 128
NCHUNK = PER_W // C
NBUF = 2
SETC = NBUF * C
NG = NCHUNK // NBUF
NPAIR = NG // 2


@functools.partial(
    pl.kernel,
    out_type=(
        jax.ShapeDtypeStruct((TOT, D), jnp.float32),
        jax.ShapeDtypeStruct((TOT, D), jnp.float32),
    ),
    mesh=plsc.VectorSubcoreMesh(core_axis_name="c", subcore_axis_name="s"),
    compiler_params=pltpu.CompilerParams(use_tc_tiling_on_sc=False),
    scratch_types=[
        pltpu.VMEM((NCHUNK, C), jnp.int32),
        pltpu.VMEM((2, SETC, D), jnp.float32),
        pltpu.VMEM((2, SETC, D), jnp.float32),
        pltpu.SemaphoreType.DMA,
        pltpu.SemaphoreType.DMA,
        pltpu.SemaphoreType.DMA,
        pltpu.SemaphoreType.DMA,
    ],
)
def _bracket_gather(idx_hbm, bra_hbm, ket_hbm, bra_out, ket_out,
                    idx_v, bra_v, ket_v, gsem0, gsem1, ssem0, ssem1):
    wid = lax.axis_index("s") * NC + lax.axis_index("c")
    base = wid * PER_W
    gsem = (gsem0, gsem1)
    ssem = (ssem0, ssem1)

    pltpu.sync_copy(idx_hbm.at[wid], idx_v)

    def fire_gathers(g, set_):
        for b in range(NBUF):
            j = g * NBUF + b
            pltpu.async_copy(
                bra_hbm.at[idx_v.at[j]], bra_v.at[set_, pl.ds(b * C, C)],
                gsem[set_])
            pltpu.async_copy(
                ket_hbm.at[idx_v.at[j]], ket_v.at[set_, pl.ds(b * C, C)],
                gsem[set_])

    def wait_gathers(set_):
        pltpu.make_async_copy(
            bra_out.at[pl.ds(0, SETC)], bra_v.at[set_], gsem[set_]).wait()
        pltpu.make_async_copy(
            ket_out.at[pl.ds(0, SETC)], ket_v.at[set_], gsem[set_]).wait()

    def fire_stores(g, set_):
        off = base + g * SETC
        pltpu.async_copy(bra_v.at[set_], bra_out.at[pl.ds(off, SETC)],
                         ssem[set_])
        pltpu.async_copy(ket_v.at[set_], ket_out.at[pl.ds(off, SETC)],
                         ssem[set_])

    def wait_stores(set_):
        pltpu.make_async_copy(
            bra_v.at[set_], bra_out.at[pl.ds(0, SETC)], ssem[set_]).wait()
        pltpu.make_async_copy(
            ket_v.at[set_], ket_out.at[pl.ds(0, SETC)], ssem[set_]).wait()

    fire_gathers(0, 0)

    def pair(p, carry):
        for parity in range(2):
            g = p * 2 + parity
            other = 1 - parity
            @pl.when(g >= 1)
            def _():
                wait_stores(other)
            @pl.when(g + 1 < NG)
            def _():
                fire_gathers(g + 1, other)
            wait_gathers(parity)
            fire_stores(g, parity)
        return carry

    lax.fori_loop(0, NPAIR, pair, 0)
    wait_stores((NG - 1) % 2)


def kernel(index, bra_weight, ket_weight):
    idx = index.reshape(NW, NCHUNK, C).astype(jnp.int32)
    bra_flat, ket_flat = _bracket_gather(idx, bra_weight, ket_weight)
    return (bra_flat.reshape(B, F, D), ket_flat.reshape(B, F, D))
